# baseline (speedup 1.0000x reference)
import functools

import numpy as np
import jax
import jax.numpy as jnp
from jax import lax
from jax.experimental import pallas as pl
from jax.experimental.pallas import tpu as pltpu
from jax.experimental.pallas import tpu_sc as plsc

N_NODES = 10000
N_EDGES = 160000
D = 256
DH = 128
DP = DH // 2
NC = 2
NS = 16
L = 16
ALPHA = 0.1

K = 80
K2 = K // 2
EDGES_PER_TILE = N_EDGES // NS
HALF_PER_TILE = EDGES_PER_TILE // 2
BLOCKS = EDGES_PER_TILE // K
ROWS_A = 624
ROWS_B = N_NODES - 15 * ROWS_A

_PERM_LO = np.concatenate([np.arange(32 * j, 32 * j + 16) for j in range(4)])
_PERM_HI = _PERM_LO + 16
_ROWS_LO = np.concatenate([c * DH + _PERM_LO for c in range(NC)])
_ROWS_HI = np.concatenate([c * DH + _PERM_HI for c in range(NC)])


def _select_row(b2d, c):
    mask = lax.broadcasted_iota(jnp.int32, b2d.shape, 0) == c
    return jnp.sum(jnp.where(mask, b2d, 0.0), axis=0, keepdims=True)


def _h_body(node_ref, w_ref, b_ref, out_ref):
    bias = _select_row(b_ref[...], pl.program_id(0))
    out_ref[0] = lax.dot_general(
        node_ref[...], w_ref[...], (((1,), (1,)), ((), ())),
        preferred_element_type=jnp.float32) + bias


def _pack16(y_lo, y_hi):
    u_lo = lax.bitcast_convert_type(y_lo.astype(jnp.bfloat16),
                                    jnp.uint16).astype(jnp.uint32)
    u_hi = lax.bitcast_convert_type(y_hi.astype(jnp.bfloat16),
                                    jnp.uint16).astype(jnp.uint32)
    return lax.bitcast_convert_type(u_lo | (u_hi << 16), jnp.int32)


def _f_body(xe_ref, xo_ref, wl_ref, wh_ref, bl_ref, bh_ref, out_ref):
    c = pl.program_id(0)
    bias_l = _select_row(bl_ref[...], c)
    bias_h = _select_row(bh_ref[...], c)

    def halfdot(x, w_ref, bias):
        return lax.dot_general(
            x, w_ref[...], (((1,), (1,)), ((), ())),
            preferred_element_type=jnp.float32) + bias

    xe, xo = xe_ref[...], xo_ref[...]
    pe = _pack16(halfdot(xe, wl_ref, bias_l), halfdot(xe, wh_ref, bias_h))
    po = _pack16(halfdot(xo, wl_ref, bias_l), halfdot(xo, wh_ref, bias_h))
    out_ref[...] = jnp.concatenate([pe, po], axis=1)


def _fin_body(node_ref, agg_ref, out_ref):
    out_ref[...] = node_ref[...] + ALPHA * jnp.maximum(agg_ref[0], 0.0)


_SC_MESH = plsc.VectorSubcoreMesh(core_axis_name="c", subcore_axis_name="s")


@functools.partial(
    pl.kernel,
    out_type=jax.ShapeDtypeStruct((NC, N_NODES, DH), jnp.float32),
    mesh=_SC_MESH,
    scratch_types=[
        pltpu.VMEM((HALF_PER_TILE,), jnp.int32),
        pltpu.VMEM((HALF_PER_TILE,), jnp.int32),
        pltpu.VMEM((2, K), jnp.int32),
        pltpu.VMEM((2, K, DH), jnp.float32),
        pltpu.VMEM((K2, 2 * DP), jnp.int32),
        pltpu.VMEM((K, DH), jnp.float32),
        pltpu.VMEM_SHARED((N_NODES, DH), jnp.float32),
        pltpu.SemaphoreType.DMA,
        pltpu.SemaphoreType.DMA,
        pltpu.SemaphoreType.DMA,
        pltpu.SemaphoreType.DMA,
        pltpu.SemaphoreType.DMA,
        pltpu.SemaphoreType.DMA,
        pltpu.SemaphoreType.DMA,
    ],
)
def _sc_edge_aggregate(ei_hbm, h2_hbm, f2_hbm, agg2_hbm,
                       srcl_v, srch_v, dst_v, h_v, f_v, m_v, agg_sh,
                       sbk, si0, si1, sg0, sg1, sfp, ssc):
    cid = lax.axis_index("c")
    sid = lax.axis_index("s")
    sis, sgs = (si0, si1), (sg0, sg1)

    pltpu.async_copy(
        ei_hbm.at[pl.ds(sid * HALF_PER_TILE, HALF_PER_TILE)], srcl_v, sbk)
    pltpu.async_copy(
        ei_hbm.at[pl.ds((NS + sid) * HALF_PER_TILE, HALF_PER_TILE)],
        srch_v, sbk)

    zero = jnp.zeros((L,), jnp.float32)

    def zrow(r, _):
        for j in range(DH // L):
            m_v[r, pl.ds(j * L, L)] = zero
        return 0

    lax.fori_loop(0, K, zrow, 0)
    row0 = sid * ROWS_A

    def _zfill(nrows):
        nfull, rem = nrows // K, nrows % K
        for i in range(nfull):
            pltpu.sync_copy(m_v, agg_sh.at[pl.ds(row0 + i * K, K)])
        if rem:
            pltpu.sync_copy(m_v.at[pl.ds(0, rem)],
                            agg_sh.at[pl.ds(row0 + nfull * K, rem)])

    @pl.when(sid < NS - 1)
    def _():
        _zfill(ROWS_A)

    @pl.when(sid == NS - 1)
    def _():
        _zfill(ROWS_B)

    for _ in range(2):
        pltpu.make_async_copy(
            ei_hbm.at[pl.ds(sid * HALF_PER_TILE, HALF_PER_TILE)],
            srcl_v, sbk).wait()
    plsc.subcore_barrier()

    htab = h2_hbm.at[cid]

    def gf_copies(g, p):
        return (
            pltpu.make_async_copy(
                htab.at[srcl_v.at[pl.ds(g * K2, K2)]],
                h_v.at[p, pl.ds(0, K2)], sgs[p]),
            pltpu.make_async_copy(
                htab.at[srch_v.at[pl.ds(g * K2, K2)]],
                h_v.at[p, pl.ds(K2, K2)], sgs[p]),
        )

    def f_copy(g):
        return pltpu.make_async_copy(
            f2_hbm.at[pl.ds(cid * (N_EDGES // 2)
                            + sid * HALF_PER_TILE + g * K2, K2)],
            f_v, sfp)

    dst0 = N_EDGES

    def dst_copies(g, p):
        return (
            pltpu.make_async_copy(
                ei_hbm.at[pl.ds(dst0 + sid * HALF_PER_TILE + g * K2, K2)],
                dst_v.at[p, pl.ds(0, K2)], sis[p]),
            pltpu.make_async_copy(
                ei_hbm.at[pl.ds(dst0 + (NS + sid) * HALF_PER_TILE + g * K2,
                                K2)],
                dst_v.at[p, pl.ds(K2, K2)], sis[p]),
        )

    for g0 in (0, 1):
        for c_ in gf_copies(g0, g0):
            c_.start()
        for c_ in dst_copies(g0, g0):
            c_.start()
    f_copy(0).start()

    himask = jnp.full((L,), -65536, jnp.int32)

    def outer(step, _):
        g0 = step * 2
        for p in range(2):
            g = g0 + p

            @pl.when(g < BLOCKS)
            def _():
                for c_ in gf_copies(g, p):
                    c_.wait()
                f_copy(g).wait()

                @pl.when(g >= 1)
                def _():
                    pltpu.make_async_copy(
                        m_v, agg_sh.at[dst_v.at[1 - p]], ssc).wait()

                @plsc.parallel_loop(0, K2, unroll=2)
                def _(t):
                    for e_off, c0 in ((0, 0), (K2, DP)):
                        r = t + e_off
                        for j in range(DP // L):
                            fi = f_v[t, pl.ds(c0 + j * L, L)]
                            fa = lax.bitcast_convert_type(
                                lax.shift_left(fi, 16), jnp.float32)
                            fb = lax.bitcast_convert_type(
                                fi & himask, jnp.float32)
                            sl_a = pl.ds(2 * j * L, L)
                            sl_b = pl.ds((2 * j + 1) * L, L)
                            m_v[r, sl_a] = h_v[p, r, sl_a] * fa
                            m_v[r, sl_b] = h_v[p, r, sl_b] * fb
                for c_ in dst_copies(g, p):
                    c_.wait()
                pltpu.make_async_copy(
                    m_v, agg_sh.at[dst_v.at[p]], ssc).start(add=True)

                @pl.when(g + 1 < BLOCKS)
                def _():
                    f_copy(g + 1).start()

                @pl.when(g + 2 < BLOCKS)
                def _():
                    for c_ in gf_copies(g + 2, p):
                        c_.start()
                    for c_ in dst_copies(g + 2, p):
                        c_.start()

        return 0

    lax.fori_loop(0, (BLOCKS + 1) // 2, outer, 0)
    pltpu.make_async_copy(
        m_v, agg_sh.at[dst_v.at[(BLOCKS - 1) % 2]], ssc).wait()
    plsc.subcore_barrier()

    @pl.when(sid < NS - 1)
    def _():
        pltpu.sync_copy(agg_sh.at[pl.ds(row0, ROWS_A)],
                        agg2_hbm.at[cid, pl.ds(row0, ROWS_A)])

    @pl.when(sid == NS - 1)
    def _():
        pltpu.sync_copy(agg_sh.at[pl.ds(row0, ROWS_B)],
                        agg2_hbm.at[cid, pl.ds(row0, ROWS_B)])


def kernel(node_feats, edge_index, edge_feats, W, b, We, be):
    ei = edge_index.astype(jnp.int32).reshape(-1)
    b2 = b.reshape(NC, DH)

    nb_h = 5
    bh = N_NODES // nb_h
    h2 = pl.pallas_call(
        _h_body,
        grid=(NC, nb_h),
        in_specs=[
            pl.BlockSpec((bh, D), lambda c, i: (i, 0)),
            pl.BlockSpec((DH, D), lambda c, i: (c, 0)),
            pl.BlockSpec((NC, DH), lambda c, i: (0, 0)),
        ],
        out_specs=pl.BlockSpec((1, bh, DH), lambda c, i: (c, i, 0)),
        out_shape=jax.ShapeDtypeStruct((NC, N_NODES, DH), jnp.float32),
    )(node_feats, W, b2)

    w_lo = We[_ROWS_LO]
    w_hi = We[_ROWS_HI]
    b_lo = be[_ROWS_LO].reshape(NC, DP)
    b_hi = be[_ROWS_HI].reshape(NC, DP)
    nb_f = 40
    bf = (N_EDGES // 2) // nb_f
    f2 = pl.pallas_call(
        _f_body,
        grid=(NC, nb_f),
        in_specs=[
            pl.BlockSpec((bf, 16), lambda c, j: (j, 0)),
            pl.BlockSpec((bf, 16), lambda c, j, _n=nb_f: (_n + j, 0)),
            pl.BlockSpec((DP, 16), lambda c, j: (c, 0)),
            pl.BlockSpec((DP, 16), lambda c, j: (c, 0)),
            pl.BlockSpec((NC, DP), lambda c, j: (0, 0)),
            pl.BlockSpec((NC, DP), lambda c, j: (0, 0)),
        ],
        out_specs=pl.BlockSpec((bf, 2 * DP), lambda c, j: (c * 40 + j, 0)),
        out_shape=jax.ShapeDtypeStruct((NC * N_EDGES // 2, 2 * DP),
                                       jnp.int32),
    )(edge_feats, edge_feats, w_lo, w_hi, b_lo, b_hi)

    agg2 = _sc_edge_aggregate(ei, h2, f2)

    out = pl.pallas_call(
        _fin_body,
        grid=(NC, nb_h),
        in_specs=[
            pl.BlockSpec((bh, DH), lambda c, i: (i, c)),
            pl.BlockSpec((1, bh, DH), lambda c, i: (c, i, 0)),
        ],
        out_specs=pl.BlockSpec((bh, DH), lambda c, i: (i, c)),
        out_shape=jax.ShapeDtypeStruct((N_NODES, D), jnp.float32),
    )(node_feats, agg2)
    return out


# speedup vs baseline: 1.0800x; 1.0800x over previous
import functools

import numpy as np
import jax
import jax.numpy as jnp
from jax import lax
from jax.experimental import pallas as pl
from jax.experimental.pallas import tpu as pltpu
from jax.experimental.pallas import tpu_sc as plsc

N_NODES = 10000
N_EDGES = 160000
D = 256
DH = 128
DP = DH // 2
NC = 2
NS = 16
L = 16
ALPHA = 0.1

K = 80
K2 = K // 2
EDGES_PER_TILE = N_EDGES // NS
HALF_PER_TILE = EDGES_PER_TILE // 2
BLOCKS = EDGES_PER_TILE // K
ROWS_A = 624
ROWS_B = N_NODES - 15 * ROWS_A

_PERM_LO = np.concatenate([np.arange(32 * j, 32 * j + 16) for j in range(4)])
_PERM_HI = _PERM_LO + 16
_ROWS_LO = np.concatenate([c * DH + _PERM_LO for c in range(NC)])
_ROWS_HI = np.concatenate([c * DH + _PERM_HI for c in range(NC)])


def _select_row(b2d, c):
    mask = lax.broadcasted_iota(jnp.int32, b2d.shape, 0) == c
    return jnp.sum(jnp.where(mask, b2d, 0.0), axis=0, keepdims=True)


def _h_body(node_ref, w_ref, b_ref, out_ref):
    bias = _select_row(b_ref[...], pl.program_id(0))
    out_ref[0] = lax.dot_general(
        node_ref[...], w_ref[...], (((1,), (1,)), ((), ())),
        preferred_element_type=jnp.float32) + bias


def _pack16(y_lo, y_hi):
    u_lo = lax.bitcast_convert_type(y_lo.astype(jnp.bfloat16),
                                    jnp.uint16).astype(jnp.uint32)
    u_hi = lax.bitcast_convert_type(y_hi.astype(jnp.bfloat16),
                                    jnp.uint16).astype(jnp.uint32)
    return lax.bitcast_convert_type(u_lo | (u_hi << 16), jnp.int32)


def _f_body(xe_ref, xo_ref, wl_ref, wh_ref, bl_ref, bh_ref, out_ref):
    c = pl.program_id(0)
    bias_l = _select_row(bl_ref[...], c)
    bias_h = _select_row(bh_ref[...], c)

    def halfdot(x, w_ref, bias):
        return lax.dot_general(
            x, w_ref[...], (((1,), (1,)), ((), ())),
            preferred_element_type=jnp.float32) + bias

    xe, xo = xe_ref[...], xo_ref[...]
    pe = _pack16(halfdot(xe, wl_ref, bias_l), halfdot(xe, wh_ref, bias_h))
    po = _pack16(halfdot(xo, wl_ref, bias_l), halfdot(xo, wh_ref, bias_h))
    out_ref[...] = jnp.concatenate([pe, po], axis=1)


def _fin_body(node_ref, agg_ref, out_ref):
    out_ref[...] = node_ref[...] + ALPHA * jnp.maximum(agg_ref[0], 0.0)


_SC_MESH = plsc.VectorSubcoreMesh(core_axis_name="c", subcore_axis_name="s")


@functools.partial(
    pl.kernel,
    out_type=jax.ShapeDtypeStruct((NC, N_NODES, DH), jnp.float32),
    mesh=_SC_MESH,
    scratch_types=[
        pltpu.VMEM((HALF_PER_TILE,), jnp.int32),
        pltpu.VMEM((HALF_PER_TILE,), jnp.int32),
        pltpu.VMEM((2, K), jnp.int32),
        pltpu.VMEM((2, K, DH), jnp.float32),
        pltpu.VMEM((K2, 2 * DP), jnp.int32),
        pltpu.VMEM((K, DH), jnp.float32),
        pltpu.VMEM_SHARED((N_NODES, DH), jnp.float32),
        pltpu.SemaphoreType.DMA,
        pltpu.SemaphoreType.DMA,
        pltpu.SemaphoreType.DMA,
        pltpu.SemaphoreType.DMA,
        pltpu.SemaphoreType.DMA,
        pltpu.SemaphoreType.DMA,
        pltpu.SemaphoreType.DMA,
    ],
)
def _sc_edge_aggregate(ei_hbm, h2_hbm, f2_hbm, agg2_hbm,
                       srcl_v, srch_v, dst_v, h_v, f_v, m_v, agg_sh,
                       sbk, si0, si1, sg0, sg1, sfp, ssc):
    cid = lax.axis_index("c")
    sid = lax.axis_index("s")
    sis, sgs = (si0, si1), (sg0, sg1)

    pltpu.async_copy(
        ei_hbm.at[pl.ds(sid * HALF_PER_TILE, HALF_PER_TILE)], srcl_v, sbk)
    pltpu.async_copy(
        ei_hbm.at[pl.ds((NS + sid) * HALF_PER_TILE, HALF_PER_TILE)],
        srch_v, sbk)

    zero = jnp.zeros((L,), jnp.float32)

    def zrow(r, _):
        for j in range(DH // L):
            m_v[r, pl.ds(j * L, L)] = zero
        return 0

    lax.fori_loop(0, K, zrow, 0)
    row0 = sid * ROWS_A

    def _zfill(nrows):
        nfull, rem = nrows // K, nrows % K
        for i in range(nfull):
            pltpu.sync_copy(m_v, agg_sh.at[pl.ds(row0 + i * K, K)])
        if rem:
            pltpu.sync_copy(m_v.at[pl.ds(0, rem)],
                            agg_sh.at[pl.ds(row0 + nfull * K, rem)])

    @pl.when(sid < NS - 1)
    def _():
        _zfill(ROWS_A)

    @pl.when(sid == NS - 1)
    def _():
        _zfill(ROWS_B)

    for _ in range(2):
        pltpu.make_async_copy(
            ei_hbm.at[pl.ds(sid * HALF_PER_TILE, HALF_PER_TILE)],
            srcl_v, sbk).wait()
    plsc.subcore_barrier()

    htab = h2_hbm.at[cid]

    def gf_copies(g, p):
        return (
            pltpu.make_async_copy(
                htab.at[srcl_v.at[pl.ds(g * K2, K2)]],
                h_v.at[p, pl.ds(0, K2)], sgs[p]),
            pltpu.make_async_copy(
                htab.at[srch_v.at[pl.ds(g * K2, K2)]],
                h_v.at[p, pl.ds(K2, K2)], sgs[p]),
        )

    def f_copy(g):
        return pltpu.make_async_copy(
            f2_hbm.at[pl.ds(cid * (N_EDGES // 2)
                            + sid * HALF_PER_TILE + g * K2, K2)],
            f_v, sfp)

    dst0 = N_EDGES

    def dst_copies(g, p):
        return (
            pltpu.make_async_copy(
                ei_hbm.at[pl.ds(dst0 + sid * HALF_PER_TILE + g * K2, K2)],
                dst_v.at[p, pl.ds(0, K2)], sis[p]),
            pltpu.make_async_copy(
                ei_hbm.at[pl.ds(dst0 + (NS + sid) * HALF_PER_TILE + g * K2,
                                K2)],
                dst_v.at[p, pl.ds(K2, K2)], sis[p]),
        )

    for g0 in (0, 1):
        for c_ in gf_copies(g0, g0):
            c_.start()
        for c_ in dst_copies(g0, g0):
            c_.start()
    f_copy(0).start()

    himask = jnp.full((L,), -65536, jnp.int32)

    def outer(step, _):
        g0 = step * 2
        for p in range(2):
            g = g0 + p

            @pl.when(g < BLOCKS)
            def _():
                for c_ in gf_copies(g, p):
                    c_.wait()
                f_copy(g).wait()

                @pl.when(g >= 1)
                def _():
                    pltpu.make_async_copy(
                        m_v, agg_sh.at[dst_v.at[1 - p]], ssc).wait()

                def mrow(t, _):
                    for e_off, c0 in ((0, 0), (K2, DP)):
                        r = t + e_off
                        for j in range(DP // L):
                            fi = f_v[t, pl.ds(c0 + j * L, L)]
                            fa = lax.bitcast_convert_type(
                                lax.shift_left(fi, 16), jnp.float32)
                            fb = lax.bitcast_convert_type(
                                fi & himask, jnp.float32)
                            sl_a = pl.ds(2 * j * L, L)
                            sl_b = pl.ds((2 * j + 1) * L, L)
                            m_v[r, sl_a] = h_v[p, r, sl_a] * fa
                            m_v[r, sl_b] = h_v[p, r, sl_b] * fb
                    return 0

                lax.fori_loop(0, K2, mrow, 0)
                for c_ in dst_copies(g, p):
                    c_.wait()
                pltpu.make_async_copy(
                    m_v, agg_sh.at[dst_v.at[p]], ssc).start(add=True)

                @pl.when(g + 1 < BLOCKS)
                def _():
                    f_copy(g + 1).start()

                @pl.when(g + 2 < BLOCKS)
                def _():
                    for c_ in gf_copies(g + 2, p):
                        c_.start()
                    for c_ in dst_copies(g + 2, p):
                        c_.start()

        return 0

    lax.fori_loop(0, (BLOCKS + 1) // 2, outer, 0)
    pltpu.make_async_copy(
        m_v, agg_sh.at[dst_v.at[(BLOCKS - 1) % 2]], ssc).wait()
    plsc.subcore_barrier()

    @pl.when(sid < NS - 1)
    def _():
        pltpu.sync_copy(agg_sh.at[pl.ds(row0, ROWS_A)],
                        agg2_hbm.at[cid, pl.ds(row0, ROWS_A)])

    @pl.when(sid == NS - 1)
    def _():
        pltpu.sync_copy(agg_sh.at[pl.ds(row0, ROWS_B)],
                        agg2_hbm.at[cid, pl.ds(row0, ROWS_B)])


def kernel(node_feats, edge_index, edge_feats, W, b, We, be):
    ei = edge_index.astype(jnp.int32).reshape(-1)
    b2 = b.reshape(NC, DH)

    nb_h = 5
    bh = N_NODES // nb_h
    h2 = pl.pallas_call(
        _h_body,
        grid=(NC, nb_h),
        in_specs=[
            pl.BlockSpec((bh, D), lambda c, i: (i, 0)),
            pl.BlockSpec((DH, D), lambda c, i: (c, 0)),
            pl.BlockSpec((NC, DH), lambda c, i: (0, 0)),
        ],
        out_specs=pl.BlockSpec((1, bh, DH), lambda c, i: (c, i, 0)),
        out_shape=jax.ShapeDtypeStruct((NC, N_NODES, DH), jnp.float32),
    )(node_feats, W, b2)

    w_lo = We[_ROWS_LO]
    w_hi = We[_ROWS_HI]
    b_lo = be[_ROWS_LO].reshape(NC, DP)
    b_hi = be[_ROWS_HI].reshape(NC, DP)
    nb_f = 20
    bf = (N_EDGES // 2) // nb_f
    f2 = pl.pallas_call(
        _f_body,
        grid=(NC, nb_f),
        in_specs=[
            pl.BlockSpec((bf, 16), lambda c, j: (j, 0)),
            pl.BlockSpec((bf, 16), lambda c, j, _n=nb_f: (_n + j, 0)),
            pl.BlockSpec((DP, 16), lambda c, j: (c, 0)),
            pl.BlockSpec((DP, 16), lambda c, j: (c, 0)),
            pl.BlockSpec((NC, DP), lambda c, j: (0, 0)),
            pl.BlockSpec((NC, DP), lambda c, j: (0, 0)),
        ],
        out_specs=pl.BlockSpec((bf, 2 * DP),
                               lambda c, j, _n=nb_f: (c * _n + j, 0)),
        out_shape=jax.ShapeDtypeStruct((NC * N_EDGES // 2, 2 * DP),
                                       jnp.int32),
    )(edge_feats, edge_feats, w_lo, w_hi, b_lo, b_hi)

    agg2 = _sc_edge_aggregate(ei, h2, f2)

    out = pl.pallas_call(
        _fin_body,
        grid=(NC, nb_h),
        in_specs=[
            pl.BlockSpec((bh, DH), lambda c, i: (i, c)),
            pl.BlockSpec((1, bh, DH), lambda c, i: (c, i, 0)),
        ],
        out_specs=pl.BlockSpec((bh, DH), lambda c, i: (i, c)),
        out_shape=jax.ShapeDtypeStruct((N_NODES, D), jnp.float32),
    )(node_feats, agg2)
    return out


# speedup vs baseline: 1.1159x; 1.0333x over previous
import functools

import numpy as np
import jax
import jax.numpy as jnp
from jax import lax
from jax.experimental import pallas as pl
from jax.experimental.pallas import tpu as pltpu
from jax.experimental.pallas import tpu_sc as plsc

N_NODES = 10000
N_EDGES = 160000
D = 256
DH = 128
DP = DH // 2
NC = 2
NS = 16
L = 16
ALPHA = 0.1

K = 80
K2 = K // 2
EDGES_PER_TILE = N_EDGES // NS
HALF_PER_TILE = EDGES_PER_TILE // 2
BLOCKS = EDGES_PER_TILE // K
ROWS_A = 624
ROWS_B = N_NODES - 15 * ROWS_A

_PERM_LO = np.concatenate([np.arange(32 * j, 32 * j + 16) for j in range(4)])
_PERM_HI = _PERM_LO + 16
_ROWS_LO = np.concatenate([c * DH + _PERM_LO for c in range(NC)])
_ROWS_HI = np.concatenate([c * DH + _PERM_HI for c in range(NC)])


def _select_row(b2d, c):
    mask = lax.broadcasted_iota(jnp.int32, b2d.shape, 0) == c
    return jnp.sum(jnp.where(mask, b2d, 0.0), axis=0, keepdims=True)


def _h_body(node_ref, w_ref, b_ref, out_ref):
    bias = _select_row(b_ref[...], pl.program_id(0))
    out_ref[0] = lax.dot_general(
        node_ref[...], w_ref[...], (((1,), (1,)), ((), ())),
        preferred_element_type=jnp.float32) + bias


def _pack16(y_lo, y_hi):
    u_lo = lax.bitcast_convert_type(y_lo.astype(jnp.bfloat16),
                                    jnp.uint16).astype(jnp.uint32)
    u_hi = lax.bitcast_convert_type(y_hi.astype(jnp.bfloat16),
                                    jnp.uint16).astype(jnp.uint32)
    return lax.bitcast_convert_type(u_lo | (u_hi << 16), jnp.int32)


def _f_body(xe_ref, xo_ref, wl_ref, wh_ref, bl_ref, bh_ref, out_ref):
    c = pl.program_id(0)
    bias_l = _select_row(bl_ref[...], c)
    bias_h = _select_row(bh_ref[...], c)

    def halfdot(x, w_ref, bias):
        return lax.dot_general(
            x, w_ref[...], (((1,), (1,)), ((), ())),
            preferred_element_type=jnp.float32) + bias

    xe, xo = xe_ref[...], xo_ref[...]
    pe = _pack16(halfdot(xe, wl_ref, bias_l), halfdot(xe, wh_ref, bias_h))
    po = _pack16(halfdot(xo, wl_ref, bias_l), halfdot(xo, wh_ref, bias_h))
    out_ref[...] = jnp.concatenate([pe, po], axis=1)


def _fin_body(node_ref, agg_ref, out_ref):
    out_ref[...] = node_ref[...] + ALPHA * jnp.maximum(agg_ref[0], 0.0)


_SC_MESH = plsc.VectorSubcoreMesh(core_axis_name="c", subcore_axis_name="s")


@functools.partial(
    pl.kernel,
    out_type=jax.ShapeDtypeStruct((NC, N_NODES, DH), jnp.float32),
    mesh=_SC_MESH,
    scratch_types=[
        pltpu.VMEM((HALF_PER_TILE,), jnp.int32),
        pltpu.VMEM((HALF_PER_TILE,), jnp.int32),
        pltpu.VMEM((2, K), jnp.int32),
        pltpu.VMEM((2, K, DH), jnp.float32),
        pltpu.VMEM((K2, 2 * DP), jnp.int32),
        pltpu.VMEM((K, DH), jnp.float32),
        pltpu.VMEM_SHARED((N_NODES, DH), jnp.float32),
        pltpu.SemaphoreType.DMA,
        pltpu.SemaphoreType.DMA,
        pltpu.SemaphoreType.DMA,
        pltpu.SemaphoreType.DMA,
        pltpu.SemaphoreType.DMA,
        pltpu.SemaphoreType.DMA,
        pltpu.SemaphoreType.DMA,
    ],
)
def _sc_edge_aggregate(ei_hbm, h2_hbm, f2_hbm, agg2_hbm,
                       srcl_v, srch_v, dst_v, h_v, f_v, m_v, agg_sh,
                       sbk, si0, si1, sg0, sg1, sfp, ssc):
    cid = lax.axis_index("c")
    sid = lax.axis_index("s")
    sis, sgs = (si0, si1), (sg0, sg1)

    pltpu.async_copy(
        ei_hbm.at[pl.ds(sid * HALF_PER_TILE, HALF_PER_TILE)], srcl_v, sbk)
    pltpu.async_copy(
        ei_hbm.at[pl.ds((NS + sid) * HALF_PER_TILE, HALF_PER_TILE)],
        srch_v, sbk)

    zero = jnp.zeros((L,), jnp.float32)

    def zrow(r, _):
        for j in range(DH // L):
            m_v[r, pl.ds(j * L, L)] = zero
        return 0

    lax.fori_loop(0, K, zrow, 0)
    row0 = sid * ROWS_A

    def _zfill(nrows):
        nfull, rem = nrows // K, nrows % K
        for i in range(nfull):
            pltpu.sync_copy(m_v, agg_sh.at[pl.ds(row0 + i * K, K)])
        if rem:
            pltpu.sync_copy(m_v.at[pl.ds(0, rem)],
                            agg_sh.at[pl.ds(row0 + nfull * K, rem)])

    @pl.when(sid < NS - 1)
    def _():
        _zfill(ROWS_A)

    @pl.when(sid == NS - 1)
    def _():
        _zfill(ROWS_B)

    for _ in range(2):
        pltpu.make_async_copy(
            ei_hbm.at[pl.ds(sid * HALF_PER_TILE, HALF_PER_TILE)],
            srcl_v, sbk).wait()
    plsc.subcore_barrier()

    htab = h2_hbm.at[cid]

    def gf_copies(g, p):
        return (
            pltpu.make_async_copy(
                htab.at[srcl_v.at[pl.ds(g * K2, K2)]],
                h_v.at[p, pl.ds(0, K2)], sgs[p]),
            pltpu.make_async_copy(
                htab.at[srch_v.at[pl.ds(g * K2, K2)]],
                h_v.at[p, pl.ds(K2, K2)], sgs[p]),
        )

    def f_copy(g):
        return pltpu.make_async_copy(
            f2_hbm.at[pl.ds(cid * (N_EDGES // 2)
                            + sid * HALF_PER_TILE + g * K2, K2)],
            f_v, sfp)

    dst0 = N_EDGES

    def dst_copies(g, p):
        return (
            pltpu.make_async_copy(
                ei_hbm.at[pl.ds(dst0 + sid * HALF_PER_TILE + g * K2, K2)],
                dst_v.at[p, pl.ds(0, K2)], sis[p]),
            pltpu.make_async_copy(
                ei_hbm.at[pl.ds(dst0 + (NS + sid) * HALF_PER_TILE + g * K2,
                                K2)],
                dst_v.at[p, pl.ds(K2, K2)], sis[p]),
        )

    for g0 in (0, 1):
        for c_ in gf_copies(g0, g0):
            c_.start()
        for c_ in dst_copies(g0, g0):
            c_.start()
    f_copy(0).start()

    himask = jnp.full((L,), -65536, jnp.int32)

    def outer(step, _):
        g0 = step * 2
        for p in range(2):
            g = g0 + p

            @pl.when(g < BLOCKS)
            def _():
                for c_ in gf_copies(g, p):
                    c_.wait()
                f_copy(g).wait()

                @pl.when(g >= 1)
                def _():
                    pltpu.make_async_copy(
                        m_v, agg_sh.at[dst_v.at[1 - p]], ssc).wait()

                def mrow(t, _):
                    for e_off, c0 in ((0, 0), (K2, DP)):
                        r = t + e_off
                        for j in range(DP // L):
                            fi = f_v[t, pl.ds(c0 + j * L, L)]
                            fa = lax.bitcast_convert_type(
                                lax.shift_left(fi, 16), jnp.float32)
                            fb = lax.bitcast_convert_type(
                                fi & himask, jnp.float32)
                            sl_a = pl.ds(2 * j * L, L)
                            sl_b = pl.ds((2 * j + 1) * L, L)
                            m_v[r, sl_a] = h_v[p, r, sl_a] * fa
                            m_v[r, sl_b] = h_v[p, r, sl_b] * fb
                    return 0

                lax.fori_loop(0, K2, mrow, 0)
                for c_ in dst_copies(g, p):
                    c_.wait()
                pltpu.make_async_copy(
                    m_v, agg_sh.at[dst_v.at[p]], ssc).start(add=True)

                @pl.when(g + 1 < BLOCKS)
                def _():
                    f_copy(g + 1).start()

                @pl.when(g + 2 < BLOCKS)
                def _():
                    for c_ in gf_copies(g + 2, p):
                        c_.start()
                    for c_ in dst_copies(g + 2, p):
                        c_.start()

        return 0

    lax.fori_loop(0, (BLOCKS + 1) // 2, outer, 0)
    pltpu.make_async_copy(
        m_v, agg_sh.at[dst_v.at[(BLOCKS - 1) % 2]], ssc).wait()
    plsc.subcore_barrier()

    @pl.when(sid < NS - 1)
    def _():
        pltpu.sync_copy(agg_sh.at[pl.ds(row0, ROWS_A)],
                        agg2_hbm.at[cid, pl.ds(row0, ROWS_A)])

    @pl.when(sid == NS - 1)
    def _():
        pltpu.sync_copy(agg_sh.at[pl.ds(row0, ROWS_B)],
                        agg2_hbm.at[cid, pl.ds(row0, ROWS_B)])


def kernel(node_feats, edge_index, edge_feats, W, b, We, be):
    ei = edge_index.astype(jnp.int32).reshape(-1)
    b2 = b.reshape(NC, DH)

    nb_h = 2
    bh = N_NODES // nb_h
    h2 = pl.pallas_call(
        _h_body,
        grid=(NC, nb_h),
        in_specs=[
            pl.BlockSpec((bh, D), lambda c, i: (i, 0)),
            pl.BlockSpec((DH, D), lambda c, i: (c, 0)),
            pl.BlockSpec((NC, DH), lambda c, i: (0, 0)),
        ],
        out_specs=pl.BlockSpec((1, bh, DH), lambda c, i: (c, i, 0)),
        out_shape=jax.ShapeDtypeStruct((NC, N_NODES, DH), jnp.float32),
    )(node_feats, W, b2)

    w_lo = We[_ROWS_LO]
    w_hi = We[_ROWS_HI]
    b_lo = be[_ROWS_LO].reshape(NC, DP)
    b_hi = be[_ROWS_HI].reshape(NC, DP)
    nb_f = 10
    bf = (N_EDGES // 2) // nb_f
    f2 = pl.pallas_call(
        _f_body,
        grid=(NC, nb_f),
        in_specs=[
            pl.BlockSpec((bf, 16), lambda c, j: (j, 0)),
            pl.BlockSpec((bf, 16), lambda c, j, _n=nb_f: (_n + j, 0)),
            pl.BlockSpec((DP, 16), lambda c, j: (c, 0)),
            pl.BlockSpec((DP, 16), lambda c, j: (c, 0)),
            pl.BlockSpec((NC, DP), lambda c, j: (0, 0)),
            pl.BlockSpec((NC, DP), lambda c, j: (0, 0)),
        ],
        out_specs=pl.BlockSpec((bf, 2 * DP),
                               lambda c, j, _n=nb_f: (c * _n + j, 0)),
        out_shape=jax.ShapeDtypeStruct((NC * N_EDGES // 2, 2 * DP),
                                       jnp.int32),
    )(edge_feats, edge_feats, w_lo, w_hi, b_lo, b_hi)

    agg2 = _sc_edge_aggregate(ei, h2, f2)

    out = pl.pallas_call(
        _fin_body,
        grid=(NC, nb_h),
        in_specs=[
            pl.BlockSpec((bh, DH), lambda c, i: (i, c)),
            pl.BlockSpec((1, bh, DH), lambda c, i: (c, i, 0)),
        ],
        out_specs=pl.BlockSpec((bh, DH), lambda c, i: (i, c)),
        out_shape=jax.ShapeDtypeStruct((N_NODES, D), jnp.float32),
    )(node_feats, agg2)
    return out
